# baseline (speedup 1.0000x reference)
import functools

import jax
import jax.numpy as jnp
from jax import lax
from jax.experimental import pallas as pl
from jax.experimental.pallas import tpu as pltpu
from jax.experimental.pallas import tpu_sc as plsc

NC = 2
NS = 16
NW = NC * NS
CW = 128
ACW = 64
SCH = 16

_mesh = functools.partial(
    plsc.VectorSubcoreMesh, core_axis_name="c", subcore_axis_name="s")
_params = pltpu.CompilerParams(needs_layout_passes=False,
                               use_tc_tiling_on_sc=False)


def _deg_body(npad, ch, dst_hbm, ew_hbm, out_hbm, dst_v, ew_v, zv, deg_sp):
  c = lax.axis_index("c")
  s = lax.axis_index("s")
  wid = s * NC + c
  nsl = npad // NS

  def zbody(i, _):
    zv[pl.ds(i * 16, 16)] = jnp.zeros((16,), jnp.float32)
    return ()

  lax.fori_loop(0, nsl // 16, zbody, ())
  pltpu.sync_copy(zv, deg_sp.at[pl.ds(s * nsl, nsl)])
  pltpu.sync_copy(dst_hbm.at[wid], dst_v)
  pltpu.sync_copy(ew_hbm.at[wid], ew_v)
  plsc.subcore_barrier()

  def body(k, _):
    pltpu.sync_copy(ew_v.at[k], deg_sp.at[dst_v.at[k]], add=True)
    return ()

  lax.fori_loop(0, ch, body, ())
  plsc.subcore_barrier()
  pltpu.sync_copy(deg_sp.at[pl.ds(s * nsl, nsl)], zv)
  pltpu.sync_copy(zv, out_hbm.at[pl.ds(c * npad + s * nsl, nsl)])


def _agg_body(n, npad, ch, d, edges_hbm, dinv_hbm, xw_hbm,
              out_hbm, e_v, norm_v, dinv_v, rows3, acc_sp,
              gsem, ssem, esem):
  c = lax.axis_index("c")
  s = lax.axis_index("s")
  wid = s * NC + c
  rsl = npad // NS
  i32 = jnp.int32

  def full16(v):
    return jnp.full((16,), v, i32)

  def zrow(r, _):
    for j in range(d // 16):
      rows3[r, pl.ds(j * 16, 16)] = jnp.zeros((16,), jnp.float32)
    return ()

  lax.fori_loop(0, ACW, zrow, ())
  for b in range(rsl // ACW):
    pltpu.sync_copy(rows3.at[pl.ds(0, ACW)],
                    acc_sp.at[pl.ds(s * rsl + b * ACW, ACW)])
  pltpu.sync_copy(dinv_hbm, dinv_v)

  def norm_block(qe, qn):
    def nrow(r, _):
      er = qe * (SCH * 3) + r * 3
      nr = qn * SCH + r
      for j in range(ACW // 16):
        sl = pl.ds(j * 16, 16)
        sidx = e_v[er, sl]
        didx = e_v[er + 1, sl]
        wbits = e_v[er + 2, sl]
        dv_s = plsc.load_gather(dinv_v, [sidx])
        dv_d = plsc.load_gather(dinv_v, [didx])
        norm_v[nr, sl] = dv_s * plsc.bitcast(wbits, jnp.float32) * dv_d
      return ()

    lax.fori_loop(0, SCH, nrow, ())

  def edge_slice(g):
    return edges_hbm.at[wid, pl.ds(g * 3, SCH * 3)]

  def gather_desc(g, p):
    sb = g // SCH
    row = lax.rem(sb, 3) * (SCH * 3) + lax.rem(g, SCH) * 3
    return pltpu.make_async_copy(
        xw_hbm.at[e_v.at[row]], rows3.at[pl.ds(p * ACW, ACW)],
        gsem.at[p])

  def scatter_desc(g, p):
    sb = g // SCH
    row = lax.rem(sb, 3) * (SCH * 3) + lax.rem(g, SCH) * 3 + 1
    return pltpu.make_async_copy(
        rows3.at[pl.ds(p * ACW, ACW)], acc_sp.at[e_v.at[row]],
        ssem.at[p])

  pltpu.sync_copy(edge_slice(0), e_v.at[pl.ds(0, SCH * 3)])
  norm_block(0, 0)
  pltpu.async_copy(edge_slice(SCH), e_v.at[pl.ds(SCH * 3, SCH * 3)], esem)
  plsc.subcore_barrier()
  gather_desc(0, 0).start()
  gather_desc(1, 1).start()

  def body(g, _):
    p = lax.rem(g, 3)
    sb = g // SCH
    lk = lax.rem(g, SCH)
    qn = lax.rem(sb, 2)
    gather_desc(g, p).wait()

    nr = qn * SCH + lk

    def row(r, _):
      nb = plsc.load_gather(norm_v, [full16(nr), full16(r)])
      br = p * ACW + r
      for j in range(d // 16):
        sl = pl.ds(j * 16, 16)
        rows3[br, sl] = rows3[br, sl] * nb
      return ()

    lax.fori_loop(0, ACW, row, ())
    scatter_desc(g, p).start(add=True)

    @pl.when(g > 0)
    def _():
      scatter_desc(g - 1, lax.rem(g + 2, 3)).wait()

    g2 = g + 2

    @pl.when(jnp.logical_and(lax.rem(g2, SCH) == 0, g2 < ch))
    def _():
      sb2 = g2 // SCH
      qe2 = lax.rem(sb2, 3)
      pltpu.make_async_copy(
          edge_slice(g2), e_v.at[pl.ds(qe2 * (SCH * 3), SCH * 3)],
          esem).wait()
      norm_block(qe2, lax.rem(sb2, 2))

      @pl.when(g2 + SCH < ch)
      def _():
        qe3 = lax.rem(sb2 + 1, 3)
        pltpu.async_copy(
            edge_slice(g2 + SCH),
            e_v.at[pl.ds(qe3 * (SCH * 3), SCH * 3)], esem)

    @pl.when(g2 < ch)
    def _():
      gather_desc(g2, lax.rem(g2, 3)).start()

    return ()

  lax.fori_loop(0, ch, body, ())
  scatter_desc(ch - 1, (ch - 1) % 3).wait()
  plsc.subcore_barrier()
  for b in range(rsl // ACW):
    off = (b % 2) * ACW
    pltpu.sync_copy(acc_sp.at[pl.ds(s * rsl + b * ACW, ACW)],
                    rows3.at[pl.ds(off, ACW)])
    pltpu.sync_copy(rows3.at[pl.ds(off, ACW)],
                    out_hbm.at[c, pl.ds(s * rsl + b * ACW, ACW)])


def _make_deg(npad, ch):
  return pl.kernel(
      functools.partial(_deg_body, npad, ch),
      out_type=jax.ShapeDtypeStruct((NC * npad,), jnp.float32),
      mesh=_mesh(),
      compiler_params=_params,
      scratch_types=[
          pltpu.VMEM((ch, CW), jnp.int32),
          pltpu.VMEM((ch, CW), jnp.float32),
          pltpu.VMEM((npad // NS,), jnp.float32),
          pltpu.VMEM_SHARED((npad,), jnp.float32),
      ])


def _make_agg(n, npad, ch, d):
  return pl.kernel(
      functools.partial(_agg_body, n, npad, ch, d),
      out_type=jax.ShapeDtypeStruct((NC, npad, d), jnp.float32),
      mesh=_mesh(),
      compiler_params=_params,
      scratch_types=[
          pltpu.VMEM((3 * SCH * 3, ACW), jnp.int32),
          pltpu.VMEM((2 * SCH, ACW), jnp.float32),
          pltpu.VMEM((npad,), jnp.float32),
          pltpu.VMEM((3 * ACW, d), jnp.float32),
          pltpu.VMEM_SHARED((npad, d), jnp.float32),
          pltpu.SemaphoreType.DMA((3,)),
          pltpu.SemaphoreType.DMA((3,)),
          pltpu.SemaphoreType.DMA,
      ])


def _dinv_tc(degp_ref, dinv_ref, sl_ref):
  deg = 1.0 + degp_ref[0] + degp_ref[1]
  dinv = jnp.where(deg > 0, lax.rsqrt(jnp.where(deg > 0, deg, 1.0)), 0.0)
  dinv_ref[...] = dinv
  sl_ref[...] = dinv * dinv


def _xw_tc(x_ref, w_ref, out_ref):
  out_ref[...] = lax.dot_general(
      x_ref[...], w_ref[...], (((1,), (0,)), ((), ())),
      preferred_element_type=jnp.float32)


def _epi_mm_tc(acc_ref, xw_ref, sl_ref, b_ref, w_ref, a_ref, h_ref, xw2_ref):
  a = a_ref[0, 0]
  h = acc_ref[0] + acc_ref[1] + xw_ref[...] * sl_ref[...] + b_ref[...]
  h = jnp.where(h >= 0, h, a * h)
  h_ref[...] = h
  xw2_ref[...] = lax.dot_general(
      h, w_ref[...], (((1,), (0,)), ((), ())),
      preferred_element_type=jnp.float32)


def _epi_tc(acc_ref, xw_ref, sl_ref, b_ref, a_ref, h_ref):
  a = a_ref[0, 0]
  h = acc_ref[0] + acc_ref[1] + xw_ref[...] * sl_ref[...] + b_ref[...]
  h_ref[...] = jnp.where(h >= 0, h, a * h)


def kernel(x, edge_index, edge_weight, W1, b1, W2, b2, a):
  n, d = x.shape
  e = edge_weight.shape[0]
  src = edge_index[0].astype(jnp.int32)
  dst = edge_index[1].astype(jnp.int32)
  ew = edge_weight.astype(jnp.float32)

  epw = -(-e // (NW * ACW * SCH)) * ACW * SCH
  ep = epw * NW
  pad = ep - e
  pad_idx = (jnp.arange(pad, dtype=jnp.int32) * 131) % n
  src_f = jnp.concatenate([src, pad_idx])
  dst_f = jnp.concatenate([dst, pad_idx])
  ew_f = jnp.concatenate([ew, jnp.zeros((pad,), jnp.float32)])
  chd = epw // CW
  dst_p = dst_f.reshape(NW, chd, CW)
  ew_p = ew_f.reshape(NW, chd, CW)
  ch = epw // ACW
  ew_bits = lax.bitcast_convert_type(ew_f, jnp.int32)
  edges = jnp.stack([src_f.reshape(NW, ch, ACW),
                     dst_f.reshape(NW, ch, ACW),
                     ew_bits.reshape(NW, ch, ACW)],
                    axis=2).reshape(NW, ch * 3, ACW)

  npad = -(-n // 256) * 256
  a2d = jnp.reshape(a, (1, 1)).astype(jnp.float32)

  degp = _make_deg(npad, chd)(dst_p, ew_p)
  degp2 = degp.reshape(NC, npad // 128, 128)
  dinv2d, sl2d = pl.pallas_call(
      _dinv_tc,
      out_shape=[jax.ShapeDtypeStruct((npad // 128, 128), jnp.float32),
                 jax.ShapeDtypeStruct((npad // 128, 128), jnp.float32)],
  )(degp2)
  dinv = dinv2d.reshape(npad)
  sl_n = sl2d.reshape(npad, 1)

  bn = 1000
  grid = n // bn
  mm = pl.pallas_call(
      _xw_tc,
      grid=(grid,),
      in_specs=[pl.BlockSpec((bn, d), lambda i: (i, 0)),
                pl.BlockSpec((d, d), lambda i: (0, 0))],
      out_specs=pl.BlockSpec((bn, d), lambda i: (i, 0)),
      out_shape=jax.ShapeDtypeStruct((n, d), jnp.float32),
  )
  xw1 = mm(x, W1)

  agg = _make_agg(n, npad, ch, d)
  acc1 = agg(edges, dinv, xw1)

  epi_mm = pl.pallas_call(
      _epi_mm_tc,
      grid=(grid,),
      in_specs=[pl.BlockSpec((NC, bn, d), lambda i: (0, i, 0)),
                pl.BlockSpec((bn, d), lambda i: (i, 0)),
                pl.BlockSpec((bn, 1), lambda i: (i, 0)),
                pl.BlockSpec((1, d), lambda i: (0, 0)),
                pl.BlockSpec((d, d), lambda i: (0, 0)),
                pl.BlockSpec((1, 1), lambda i: (0, 0))],
      out_specs=[pl.BlockSpec((bn, d), lambda i: (i, 0)),
                 pl.BlockSpec((bn, d), lambda i: (i, 0))],
      out_shape=[jax.ShapeDtypeStruct((n, d), jnp.float32),
                 jax.ShapeDtypeStruct((n, d), jnp.float32)],
  )
  h1, xw2 = epi_mm(acc1, xw1, sl_n, b1.reshape(1, d), W2, a2d)

  acc2 = agg(edges, dinv, xw2)

  epi = pl.pallas_call(
      _epi_tc,
      grid=(grid,),
      in_specs=[pl.BlockSpec((NC, bn, d), lambda i: (0, i, 0)),
                pl.BlockSpec((bn, d), lambda i: (i, 0)),
                pl.BlockSpec((bn, 1), lambda i: (i, 0)),
                pl.BlockSpec((1, d), lambda i: (0, 0)),
                pl.BlockSpec((1, 1), lambda i: (0, 0))],
      out_specs=pl.BlockSpec((bn, d), lambda i: (i, 0)),
      out_shape=jax.ShapeDtypeStruct((n, d), jnp.float32),
  )
  h2 = epi(acc2, xw2, sl_n, b2.reshape(1, d), a2d)
  return h1, h2


# speedup vs baseline: 2.0849x; 2.0849x over previous
import functools

import jax
import jax.numpy as jnp
from jax import lax
from jax.experimental import pallas as pl
from jax.experimental.pallas import tpu as pltpu
from jax.experimental.pallas import tpu_sc as plsc

NC = 2
NS = 16
NW = NC * NS
CW = 128
SCH = 8

_mesh = functools.partial(
    plsc.VectorSubcoreMesh, core_axis_name="c", subcore_axis_name="s")
_params = pltpu.CompilerParams(needs_layout_passes=False,
                               use_tc_tiling_on_sc=False)


def _deg_body(npad, ch, dst_hbm, ew_hbm, out_hbm, dst_v, ew_v, zv, deg_sp):
  c = lax.axis_index("c")
  s = lax.axis_index("s")
  wid = s * NC + c
  nsl = npad // NS

  def zbody(i, _):
    zv[pl.ds(i * 16, 16)] = jnp.zeros((16,), jnp.float32)
    return ()

  lax.fori_loop(0, nsl // 16, zbody, ())
  pltpu.sync_copy(zv, deg_sp.at[pl.ds(s * nsl, nsl)])
  pltpu.sync_copy(dst_hbm.at[wid], dst_v)
  pltpu.sync_copy(ew_hbm.at[wid], ew_v)
  plsc.subcore_barrier()

  def body(k, _):
    pltpu.sync_copy(ew_v.at[k], deg_sp.at[dst_v.at[k]], add=True)
    return ()

  lax.fori_loop(0, ch, body, ())
  plsc.subcore_barrier()
  pltpu.sync_copy(deg_sp.at[pl.ds(s * nsl, nsl)], zv)
  pltpu.sync_copy(zv, out_hbm.at[pl.ds(c * npad + s * nsl, nsl)])


def _agg_body(n, npad, ch, d, src_hbm, dst_hbm, ew_hbm, dinv_hbm, xw_hbm,
              out_hbm, src_v, dst_v, ew_v, norm_v, dinv_v,
              rows_a, rows_b, acc_sp, sem_a, sem_b, sem_sa, sem_sb):
  c = lax.axis_index("c")
  s = lax.axis_index("s")
  wid = s * NC + c
  rsl = npad // NS

  def zrow(r, _):
    for j in range(d // 16):
      rows_a[r, pl.ds(j * 16, 16)] = jnp.zeros((16,), jnp.float32)
    return ()

  lax.fori_loop(0, CW, zrow, ())
  for b in range(rsl // CW):
    pltpu.sync_copy(rows_a, acc_sp.at[pl.ds(s * rsl + b * CW, CW)])
  pltpu.sync_copy(dinv_hbm, dinv_v)
  plsc.subcore_barrier()

  def scale_rows(buf, k):
    def row(r, _):
      nb = plsc.load_gather(
          norm_v, [jnp.full((16,), k, jnp.int32),
                   jnp.full((16,), r, jnp.int32)])
      for j in range(d // 16):
        buf[r, pl.ds(j * 16, 16)] = buf[r, pl.ds(j * 16, 16)] * nb
      return ()

    lax.fori_loop(0, CW, row, ())

  def scat_desc(buf, k, sem):
    return pltpu.make_async_copy(buf, acc_sp.at[dst_v.at[k]], sem)

  def super_chunk(sb, _):
    @pl.when(sb > 0)
    def _():
      scat_desc(rows_b, SCH - 1, sem_sb).wait()

    pltpu.sync_copy(src_hbm.at[wid, pl.ds(sb * SCH, SCH)], src_v)
    pltpu.sync_copy(dst_hbm.at[wid, pl.ds(sb * SCH, SCH)], dst_v)
    pltpu.sync_copy(ew_hbm.at[wid, pl.ds(sb * SCH, SCH)], ew_v)

    def norm_row(r, _):
      for j in range(CW // 16):
        sidx = src_v[r, pl.ds(j * 16, 16)]
        didx = dst_v[r, pl.ds(j * 16, 16)]
        dv_s = plsc.load_gather(dinv_v, [sidx])
        dv_d = plsc.load_gather(dinv_v, [didx])
        norm_v[r, pl.ds(j * 16, 16)] = (
            dv_s * ew_v[r, pl.ds(j * 16, 16)] * dv_d)
      return ()

    lax.fori_loop(0, SCH, norm_row, ())

    pltpu.async_copy(xw_hbm.at[src_v.at[0]], rows_a, sem_a)

    def pair(m, _):
      ka = 2 * m
      kb = ka + 1
      pltpu.make_async_copy(xw_hbm.at[src_v.at[ka]], rows_a, sem_a).wait()

      @pl.when(m > 0)
      def _():
        scat_desc(rows_b, kb - 2, sem_sb).wait()

      pltpu.async_copy(xw_hbm.at[src_v.at[kb]], rows_b, sem_b)
      scale_rows(rows_a, ka)
      scat_desc(rows_a, ka, sem_sa).start(add=True)
      pltpu.make_async_copy(xw_hbm.at[src_v.at[kb]], rows_b, sem_b).wait()
      scale_rows(rows_b, kb)
      scat_desc(rows_a, ka, sem_sa).wait()

      @pl.when(kb + 1 < SCH)
      def _():
        pltpu.async_copy(xw_hbm.at[src_v.at[kb + 1]], rows_a, sem_a)

      scat_desc(rows_b, kb, sem_sb).start(add=True)
      return ()

    lax.fori_loop(0, SCH // 2, pair, ())
    return ()

  lax.fori_loop(0, ch // SCH, super_chunk, ())
  scat_desc(rows_b, SCH - 1, sem_sb).wait()
  plsc.subcore_barrier()
  for b in range(rsl // CW):
    buf = rows_a if b % 2 == 0 else rows_b
    pltpu.sync_copy(acc_sp.at[pl.ds(s * rsl + b * CW, CW)], buf)
    pltpu.sync_copy(buf, out_hbm.at[c, pl.ds(s * rsl + b * CW, CW)])


def _make_deg(npad, ch):
  return pl.kernel(
      functools.partial(_deg_body, npad, ch),
      out_type=jax.ShapeDtypeStruct((NC * npad,), jnp.float32),
      mesh=_mesh(),
      compiler_params=_params,
      scratch_types=[
          pltpu.VMEM((ch, CW), jnp.int32),
          pltpu.VMEM((ch, CW), jnp.float32),
          pltpu.VMEM((npad // NS,), jnp.float32),
          pltpu.VMEM_SHARED((npad,), jnp.float32),
      ])


def _make_agg(n, npad, ch, d):
  return pl.kernel(
      functools.partial(_agg_body, n, npad, ch, d),
      out_type=jax.ShapeDtypeStruct((NC, npad, d), jnp.float32),
      mesh=_mesh(),
      compiler_params=_params,
      scratch_types=[
          pltpu.VMEM((SCH, CW), jnp.int32),
          pltpu.VMEM((SCH, CW), jnp.int32),
          pltpu.VMEM((SCH, CW), jnp.float32),
          pltpu.VMEM((SCH, CW), jnp.float32),
          pltpu.VMEM((npad,), jnp.float32),
          pltpu.VMEM((CW, d), jnp.float32),
          pltpu.VMEM((CW, d), jnp.float32),
          pltpu.VMEM_SHARED((npad, d), jnp.float32),
          pltpu.SemaphoreType.DMA,
          pltpu.SemaphoreType.DMA,
          pltpu.SemaphoreType.DMA,
          pltpu.SemaphoreType.DMA,
      ])


def _dinv_tc(degp_ref, dinv_ref, sl_ref):
  deg = 1.0 + degp_ref[0] + degp_ref[1]
  dinv = jnp.where(deg > 0, lax.rsqrt(jnp.where(deg > 0, deg, 1.0)), 0.0)
  dinv_ref[...] = dinv
  sl_ref[...] = dinv * dinv


def _xw_tc(x_ref, w_ref, out_ref):
  out_ref[...] = lax.dot_general(
      x_ref[...], w_ref[...], (((1,), (0,)), ((), ())),
      preferred_element_type=jnp.float32)


def _epi_mm_tc(acc_ref, xw_ref, sl_ref, b_ref, w_ref, a_ref, h_ref, xw2_ref):
  a = a_ref[0, 0]
  h = acc_ref[0] + acc_ref[1] + xw_ref[...] * sl_ref[...] + b_ref[...]
  h = jnp.where(h >= 0, h, a * h)
  h_ref[...] = h
  xw2_ref[...] = lax.dot_general(
      h, w_ref[...], (((1,), (0,)), ((), ())),
      preferred_element_type=jnp.float32)


def _epi_tc(acc_ref, xw_ref, sl_ref, b_ref, a_ref, h_ref):
  a = a_ref[0, 0]
  h = acc_ref[0] + acc_ref[1] + xw_ref[...] * sl_ref[...] + b_ref[...]
  h_ref[...] = jnp.where(h >= 0, h, a * h)


def kernel(x, edge_index, edge_weight, W1, b1, W2, b2, a):
  n, d = x.shape
  e = edge_weight.shape[0]
  src = edge_index[0].astype(jnp.int32)
  dst = edge_index[1].astype(jnp.int32)
  ew = edge_weight.astype(jnp.float32)

  epw = -(-e // (NW * CW * SCH)) * CW * SCH
  ch = epw // CW
  ep = epw * NW
  pad = ep - e
  pad_idx = (jnp.arange(pad, dtype=jnp.int32) * 131) % n
  src_p = jnp.concatenate([src, pad_idx]).reshape(NW, ch, CW)
  dst_p = jnp.concatenate([dst, pad_idx]).reshape(NW, ch, CW)
  ew_p = jnp.concatenate([ew, jnp.zeros((pad,), jnp.float32)]
                         ).reshape(NW, ch, CW)

  npad = -(-n // 256) * 256
  a2d = jnp.reshape(a, (1, 1)).astype(jnp.float32)

  degp = _make_deg(npad, ch)(dst_p, ew_p)
  degp2 = degp.reshape(NC, npad // 128, 128)
  dinv2d, sl2d = pl.pallas_call(
      _dinv_tc,
      out_shape=[jax.ShapeDtypeStruct((npad // 128, 128), jnp.float32),
                 jax.ShapeDtypeStruct((npad // 128, 128), jnp.float32)],
  )(degp2)
  dinv = dinv2d.reshape(npad)
  sl_n = sl2d.reshape(npad, 1)

  bn = 1000
  grid = n // bn
  mm = pl.pallas_call(
      _xw_tc,
      grid=(grid,),
      in_specs=[pl.BlockSpec((bn, d), lambda i: (i, 0)),
                pl.BlockSpec((d, d), lambda i: (0, 0))],
      out_specs=pl.BlockSpec((bn, d), lambda i: (i, 0)),
      out_shape=jax.ShapeDtypeStruct((n, d), jnp.float32),
  )
  xw1 = mm(x, W1)

  agg = _make_agg(n, npad, ch, d)
  acc1 = agg(src_p, dst_p, ew_p, dinv, xw1)

  epi_mm = pl.pallas_call(
      _epi_mm_tc,
      grid=(grid,),
      in_specs=[pl.BlockSpec((NC, bn, d), lambda i: (0, i, 0)),
                pl.BlockSpec((bn, d), lambda i: (i, 0)),
                pl.BlockSpec((bn, 1), lambda i: (i, 0)),
                pl.BlockSpec((1, d), lambda i: (0, 0)),
                pl.BlockSpec((d, d), lambda i: (0, 0)),
                pl.BlockSpec((1, 1), lambda i: (0, 0))],
      out_specs=[pl.BlockSpec((bn, d), lambda i: (i, 0)),
                 pl.BlockSpec((bn, d), lambda i: (i, 0))],
      out_shape=[jax.ShapeDtypeStruct((n, d), jnp.float32),
                 jax.ShapeDtypeStruct((n, d), jnp.float32)],
  )
  h1, xw2 = epi_mm(acc1, xw1, sl_n, b1.reshape(1, d), W2, a2d)

  acc2 = agg(src_p, dst_p, ew_p, dinv, xw2)

  epi = pl.pallas_call(
      _epi_tc,
      grid=(grid,),
      in_specs=[pl.BlockSpec((NC, bn, d), lambda i: (0, i, 0)),
                pl.BlockSpec((bn, d), lambda i: (i, 0)),
                pl.BlockSpec((bn, 1), lambda i: (i, 0)),
                pl.BlockSpec((1, d), lambda i: (0, 0)),
                pl.BlockSpec((1, 1), lambda i: (0, 0))],
      out_specs=pl.BlockSpec((bn, d), lambda i: (i, 0)),
      out_shape=jax.ShapeDtypeStruct((n, d), jnp.float32),
  )
  h2 = epi(acc2, xw2, sl_n, b2.reshape(1, d), a2d)
  return h1, h2


# speedup vs baseline: 2.2531x; 1.0807x over previous
import functools

import jax
import jax.numpy as jnp
from jax import lax
from jax.experimental import pallas as pl
from jax.experimental.pallas import tpu as pltpu
from jax.experimental.pallas import tpu_sc as plsc

NC = 2
NS = 16
NW = NC * NS
CW = 128
ACW = 80
SCH = 12

_mesh = functools.partial(
    plsc.VectorSubcoreMesh, core_axis_name="c", subcore_axis_name="s")
_params = pltpu.CompilerParams(needs_layout_passes=False,
                               use_tc_tiling_on_sc=False)


def _deg_body(npad, ch, dst_hbm, ew_hbm, out_hbm, dst_v, ew_v, zv, deg_sp):
  c = lax.axis_index("c")
  s = lax.axis_index("s")
  wid = s * NC + c
  nsl = npad // NS

  def zbody(i, _):
    zv[pl.ds(i * 16, 16)] = jnp.zeros((16,), jnp.float32)
    return ()

  lax.fori_loop(0, nsl // 16, zbody, ())
  pltpu.sync_copy(zv, deg_sp.at[pl.ds(s * nsl, nsl)])
  pltpu.sync_copy(dst_hbm.at[wid], dst_v)
  pltpu.sync_copy(ew_hbm.at[wid], ew_v)
  plsc.subcore_barrier()

  def body(k, _):
    pltpu.sync_copy(ew_v.at[k], deg_sp.at[dst_v.at[k]], add=True)
    return ()

  lax.fori_loop(0, ch, body, ())
  plsc.subcore_barrier()
  pltpu.sync_copy(deg_sp.at[pl.ds(s * nsl, nsl)], zv)
  pltpu.sync_copy(zv, out_hbm.at[pl.ds(c * npad + s * nsl, nsl)])


def _agg_body(n, npad, ch, d, src_hbm, dst_hbm, ew_hbm, dinv_hbm, xw_hbm,
              out_hbm, src_v, dst_v, ew_v, norm_v, dinv_v,
              rows_a, rows_b, rows_c, acc_sp,
              ga, gb, gc, sa, sb_, sc_):
  c = lax.axis_index("c")
  s = lax.axis_index("s")
  wid = s * NC + c
  rsl = npad // NS
  bufs = (rows_a, rows_b, rows_c)
  gsems = (ga, gb, gc)
  ssems = (sa, sb_, sc_)

  def zrow(r, _):
    for j in range(d // 16):
      rows_a[r, pl.ds(j * 16, 16)] = jnp.zeros((16,), jnp.float32)
    return ()

  lax.fori_loop(0, ACW, zrow, ())
  for b in range(rsl // ACW):
    pltpu.sync_copy(rows_a, acc_sp.at[pl.ds(s * rsl + b * ACW, ACW)])
  pltpu.sync_copy(dinv_hbm, dinv_v)
  plsc.subcore_barrier()

  def scale_rows(buf, k):
    def row(r, _):
      nb = plsc.load_gather(
          norm_v, [jnp.full((16,), k, jnp.int32),
                   jnp.full((16,), r, jnp.int32)])
      for j in range(d // 16):
        buf[r, pl.ds(j * 16, 16)] = buf[r, pl.ds(j * 16, 16)] * nb
      return ()

    lax.fori_loop(0, ACW, row, ())

  def gat(k, b):
    return pltpu.make_async_copy(xw_hbm.at[src_v.at[k]], bufs[b], gsems[b])

  def scat(k, b):
    return pltpu.make_async_copy(bufs[b], acc_sp.at[dst_v.at[k]], ssems[b])

  def super_chunk(sb, _):
    pltpu.sync_copy(src_hbm.at[wid, pl.ds(sb * SCH, SCH)], src_v)
    pltpu.sync_copy(dst_hbm.at[wid, pl.ds(sb * SCH, SCH)], dst_v)
    pltpu.sync_copy(ew_hbm.at[wid, pl.ds(sb * SCH, SCH)], ew_v)

    def norm_row(r, _):
      for j in range(ACW // 16):
        sidx = src_v[r, pl.ds(j * 16, 16)]
        didx = dst_v[r, pl.ds(j * 16, 16)]
        dv_s = plsc.load_gather(dinv_v, [sidx])
        dv_d = plsc.load_gather(dinv_v, [didx])
        norm_v[r, pl.ds(j * 16, 16)] = (
            dv_s * ew_v[r, pl.ds(j * 16, 16)] * dv_d)
      return ()

    lax.fori_loop(0, SCH, norm_row, ())

    gat(0, 0).start()
    gat(1, 1).start()
    for k in range(SCH):
      this = k % 3
      other = (k + 2) % 3
      gat(k, this).wait()
      scale_rows(bufs[this], k)
      if k >= 1:
        scat(k - 1, other).wait()
      if k + 2 < SCH:
        gat(k + 2, other).start()
      scat(k, this).start(add=True)
    scat(SCH - 1, (SCH - 1) % 3).wait()
    return ()

  lax.fori_loop(0, ch // SCH, super_chunk, ())
  plsc.subcore_barrier()
  for b in range(rsl // ACW):
    buf = bufs[b % 3]
    pltpu.sync_copy(acc_sp.at[pl.ds(s * rsl + b * ACW, ACW)], buf)
    pltpu.sync_copy(buf, out_hbm.at[c, pl.ds(s * rsl + b * ACW, ACW)])


def _make_deg(npad, ch):
  return pl.kernel(
      functools.partial(_deg_body, npad, ch),
      out_type=jax.ShapeDtypeStruct((NC * npad,), jnp.float32),
      mesh=_mesh(),
      compiler_params=_params,
      scratch_types=[
          pltpu.VMEM((ch, CW), jnp.int32),
          pltpu.VMEM((ch, CW), jnp.float32),
          pltpu.VMEM((npad // NS,), jnp.float32),
          pltpu.VMEM_SHARED((npad,), jnp.float32),
      ])


def _make_agg(n, npad, ch, d):
  return pl.kernel(
      functools.partial(_agg_body, n, npad, ch, d),
      out_type=jax.ShapeDtypeStruct((NC, npad, d), jnp.float32),
      mesh=_mesh(),
      compiler_params=_params,
      scratch_types=[
          pltpu.VMEM((SCH, ACW), jnp.int32),
          pltpu.VMEM((SCH, ACW), jnp.int32),
          pltpu.VMEM((SCH, ACW), jnp.float32),
          pltpu.VMEM((SCH, ACW), jnp.float32),
          pltpu.VMEM((npad,), jnp.float32),
          pltpu.VMEM((ACW, d), jnp.float32),
          pltpu.VMEM((ACW, d), jnp.float32),
          pltpu.VMEM((ACW, d), jnp.float32),
          pltpu.VMEM_SHARED((npad, d), jnp.float32),
          pltpu.SemaphoreType.DMA,
          pltpu.SemaphoreType.DMA,
          pltpu.SemaphoreType.DMA,
          pltpu.SemaphoreType.DMA,
          pltpu.SemaphoreType.DMA,
          pltpu.SemaphoreType.DMA,
      ])


def _dinv_tc(degp_ref, dinv_ref, sl_ref):
  deg = 1.0 + degp_ref[0] + degp_ref[1]
  dinv = jnp.where(deg > 0, lax.rsqrt(jnp.where(deg > 0, deg, 1.0)), 0.0)
  dinv_ref[...] = dinv
  sl_ref[...] = dinv * dinv


def _xw_tc(x_ref, w_ref, out_ref):
  out_ref[...] = lax.dot_general(
      x_ref[...], w_ref[...], (((1,), (0,)), ((), ())),
      preferred_element_type=jnp.float32)


def _epi_mm_tc(acc_ref, xw_ref, sl_ref, b_ref, w_ref, a_ref, h_ref, xw2_ref):
  a = a_ref[0, 0]
  h = acc_ref[0] + acc_ref[1] + xw_ref[...] * sl_ref[...] + b_ref[...]
  h = jnp.where(h >= 0, h, a * h)
  h_ref[...] = h
  xw2_ref[...] = lax.dot_general(
      h, w_ref[...], (((1,), (0,)), ((), ())),
      preferred_element_type=jnp.float32)


def _epi_tc(acc_ref, xw_ref, sl_ref, b_ref, a_ref, h_ref):
  a = a_ref[0, 0]
  h = acc_ref[0] + acc_ref[1] + xw_ref[...] * sl_ref[...] + b_ref[...]
  h_ref[...] = jnp.where(h >= 0, h, a * h)


def kernel(x, edge_index, edge_weight, W1, b1, W2, b2, a):
  n, d = x.shape
  e = edge_weight.shape[0]
  src = edge_index[0].astype(jnp.int32)
  dst = edge_index[1].astype(jnp.int32)
  ew = edge_weight.astype(jnp.float32)

  def padded(arrs, lanes, pad_to):
    epw = -(-e // (NW * pad_to)) * pad_to
    pad = epw * NW - e
    pidx = (jnp.arange(pad, dtype=jnp.int32) * 131) % n
    pz = jnp.zeros((pad,), jnp.float32)
    out = []
    for arr in arrs:
      fill = pz if arr.dtype == jnp.float32 else pidx
      out.append(jnp.concatenate([arr, fill])
                 .reshape(NW, epw // lanes, lanes))
    return out, epw // lanes

  (dst_p, ew_p), chd = padded([dst, ew], CW, CW)
  (src_a, dst_a, ew_a), ch = padded([src, dst, ew], ACW, ACW * SCH)

  npad = -(-n // 256) * 256
  a2d = jnp.reshape(a, (1, 1)).astype(jnp.float32)

  degp = _make_deg(npad, chd)(dst_p, ew_p)
  degp2 = degp.reshape(NC, npad // 128, 128)
  dinv2d, sl2d = pl.pallas_call(
      _dinv_tc,
      out_shape=[jax.ShapeDtypeStruct((npad // 128, 128), jnp.float32),
                 jax.ShapeDtypeStruct((npad // 128, 128), jnp.float32)],
  )(degp2)
  dinv = dinv2d.reshape(npad)
  sl_n = sl2d.reshape(npad, 1)

  bn = 1000
  grid = n // bn
  mm = pl.pallas_call(
      _xw_tc,
      grid=(grid,),
      in_specs=[pl.BlockSpec((bn, d), lambda i: (i, 0)),
                pl.BlockSpec((d, d), lambda i: (0, 0))],
      out_specs=pl.BlockSpec((bn, d), lambda i: (i, 0)),
      out_shape=jax.ShapeDtypeStruct((n, d), jnp.float32),
  )
  xw1 = mm(x, W1)

  agg = _make_agg(n, npad, ch, d)
  acc1 = agg(src_a, dst_a, ew_a, dinv, xw1)

  epi_mm = pl.pallas_call(
      _epi_mm_tc,
      grid=(grid,),
      in_specs=[pl.BlockSpec((NC, bn, d), lambda i: (0, i, 0)),
                pl.BlockSpec((bn, d), lambda i: (i, 0)),
                pl.BlockSpec((bn, 1), lambda i: (i, 0)),
                pl.BlockSpec((1, d), lambda i: (0, 0)),
                pl.BlockSpec((d, d), lambda i: (0, 0)),
                pl.BlockSpec((1, 1), lambda i: (0, 0))],
      out_specs=[pl.BlockSpec((bn, d), lambda i: (i, 0)),
                 pl.BlockSpec((bn, d), lambda i: (i, 0))],
      out_shape=[jax.ShapeDtypeStruct((n, d), jnp.float32),
                 jax.ShapeDtypeStruct((n, d), jnp.float32)],
  )
  h1, xw2 = epi_mm(acc1, xw1, sl_n, b1.reshape(1, d), W2, a2d)

  acc2 = agg(src_a, dst_a, ew_a, dinv, xw2)

  epi = pl.pallas_call(
      _epi_tc,
      grid=(grid,),
      in_specs=[pl.BlockSpec((NC, bn, d), lambda i: (0, i, 0)),
                pl.BlockSpec((bn, d), lambda i: (i, 0)),
                pl.BlockSpec((bn, 1), lambda i: (i, 0)),
                pl.BlockSpec((1, d), lambda i: (0, 0)),
                pl.BlockSpec((1, 1), lambda i: (0, 0))],
      out_specs=pl.BlockSpec((bn, d), lambda i: (i, 0)),
      out_shape=jax.ShapeDtypeStruct((n, d), jnp.float32),
  )
  h2 = epi(acc2, xw2, sl_n, b2.reshape(1, d), a2d)
  return h1, h2
